# baseline (speedup 1.0000x reference)
import functools

import jax
import jax.numpy as jnp
from jax import lax
from jax.experimental import pallas as pl
from jax.experimental.pallas import tpu as pltpu
from jax.experimental.pallas import tpu_sc as plsc

INPUT_DIM = 1024
HIDDEN_DIM = 2048
OUTPUT_DIM = 1024
NUM_EXPERTS = 8
TOP_K = 2
BATCH = 4096

TILE = 256
NUM_TILES = (BATCH * TOP_K) // TILE + (NUM_EXPERTS - 1)
NUM_TILES = ((NUM_TILES + 7) // 8) * 8
SLOTS = NUM_TILES * TILE

NC = 2
NS = 16
NW = NC * NS
L = 16
TPW = BATCH // NW
GRP = TPW // L

_SC_MESH = plsc.VectorSubcoreMesh(core_axis_name="c", subcore_axis_name="s")


def _wid():
    return lax.axis_index("s") * NC + lax.axis_index("c")


@functools.partial(
    pl.kernel,
    out_type=(
        jax.ShapeDtypeStruct((TOP_K, BATCH), jnp.int32),
        jax.ShapeDtypeStruct((TOP_K, BATCH), jnp.float32),
        jax.ShapeDtypeStruct((NW, L), jnp.int32),
        jax.ShapeDtypeStruct((NW, L), jnp.float32),
    ),
    mesh=_SC_MESH,
    compiler_params=pltpu.CompilerParams(needs_layout_passes=False),
    scratch_types=[
        pltpu.VMEM((NUM_EXPERTS, TPW), jnp.float32),
        pltpu.VMEM((TPW,), jnp.int32),
        pltpu.VMEM((TPW,), jnp.int32),
        pltpu.VMEM((TPW,), jnp.float32),
        pltpu.VMEM((TPW,), jnp.float32),
        pltpu.VMEM((L,), jnp.int32),
        pltpu.VMEM((L,), jnp.float32),
    ],
)
def _router_a(gwT, e12, wtk, counts, csums,
              gw_v, e1_v, e2_v, w0_v, w1_v, cnt_v, sv_v):
    wid = _wid()
    t0 = wid * TPW
    pltpu.sync_copy(gwT.at[:, pl.ds(t0, TPW)], gw_v)
    lanes = lax.iota(jnp.int32, L)
    cnt = jnp.zeros((L,), jnp.int32)
    sv = jnp.zeros((L,), jnp.float32)
    ones_f = jnp.full((L,), 1.0, jnp.float32)
    neg_f = jnp.full((L,), -1.0, jnp.float32)
    zeros_i = jnp.zeros((L,), jnp.int32)
    zeros_f = jnp.zeros((L,), jnp.float32)
    ev_const = [jnp.full((L,), e, jnp.int32) for e in range(NUM_EXPERTS)]
    for g in range(GRP):
        sl = pl.ds(g * L, L)
        le = [gw_v[e, sl] for e in range(NUM_EXPERTS)]
        m1 = le[0]
        for e in range(1, NUM_EXPERTS):
            m1 = jnp.maximum(m1, le[e])
        e1 = zeros_i
        for e in range(NUM_EXPERTS - 1, -1, -1):
            e1 = jnp.where(le[e] == m1, ev_const[e], e1)
        le2 = [jnp.where(e1 == ev_const[e], neg_f, le[e])
               for e in range(NUM_EXPERTS)]
        m2 = le2[0]
        for e in range(1, NUM_EXPERTS):
            m2 = jnp.maximum(m2, le2[e])
        e2 = zeros_i
        for e in range(NUM_EXPERTS - 1, -1, -1):
            e2 = jnp.where(le2[e] == m2, ev_const[e], e2)
        z = jnp.exp(m2 - m1)
        w0 = ones_f / (ones_f + z)
        e1_v[sl] = e1
        e2_v[sl] = e2
        w0_v[sl] = w0
        w1_v[sl] = z * w0
        for e in range(NUM_EXPERTS):
            c = (jnp.sum((e1 == ev_const[e]).astype(jnp.int32))
                 + jnp.sum((e2 == ev_const[e]).astype(jnp.int32)))
            cnt = cnt + jnp.where(lanes == ev_const[e],
                                  jnp.broadcast_to(c, (L,)), zeros_i)
            sv = sv + jnp.where(lanes == ev_const[e],
                                jnp.broadcast_to(jnp.sum(le[e]), (L,)),
                                zeros_f)
    cnt_v[...] = cnt
    sv_v[...] = sv
    pltpu.sync_copy(e1_v, e12.at[0, pl.ds(t0, TPW)])
    pltpu.sync_copy(e2_v, e12.at[1, pl.ds(t0, TPW)])
    pltpu.sync_copy(w0_v, wtk.at[0, pl.ds(t0, TPW)])
    pltpu.sync_copy(w1_v, wtk.at[1, pl.ds(t0, TPW)])
    pltpu.sync_copy(cnt_v, counts.at[wid])
    pltpu.sync_copy(sv_v, csums.at[wid])


@functools.partial(
    pl.kernel,
    out_type=(
        jax.ShapeDtypeStruct((TOP_K, BATCH), jnp.int32),
        jax.ShapeDtypeStruct((NUM_TILES + 8, ), jnp.int32),
        jax.ShapeDtypeStruct((L,), jnp.float32),
        jax.ShapeDtypeStruct((SLOTS, INPUT_DIM), jnp.float32),
    ),
    mesh=_SC_MESH,
    compiler_params=pltpu.CompilerParams(needs_layout_passes=False),
    scratch_types=[
        pltpu.VMEM((NW, L), jnp.int32),
        pltpu.VMEM((NW, L), jnp.float32),
        pltpu.VMEM((TOP_K, TPW), jnp.int32),
        pltpu.VMEM((TPW // 2,), jnp.int32),
        pltpu.VMEM((TPW // 2,), jnp.int32),
        pltpu.VMEM((TPW // 2,), jnp.int32),
        pltpu.VMEM((TPW // 2,), jnp.int32),
        pltpu.VMEM((TPW // 2, INPUT_DIM), jnp.float32),
        pltpu.VMEM((NUM_TILES + 8,), jnp.int32),
        pltpu.VMEM((L,), jnp.float32),
        pltpu.SemaphoreType.DMA,
    ],
)
def _router_b(e12, counts, csums, xb, pos, meta, cvv, xs,
              cnt_v, cs_v, e12_v, p0a, p0b, p1a, p1b, x_v, m_v, cv_v, sem):
    wid = _wid()
    t0 = wid * TPW
    lanes = lax.iota(jnp.int32, L)
    wid_v = jnp.broadcast_to(wid, (L,))

    HALF = TPW // 2
    cpx = pltpu.async_copy(xb.at[pl.ds(t0, HALF)], x_v, sem)
    pltpu.sync_copy(counts, cnt_v)
    total = jnp.zeros((L,), jnp.int32)
    pre = jnp.zeros((L,), jnp.int32)
    for w in range(NW):
        cw = cnt_v[w]
        total = total + cw
        pre = pre + jnp.where(jnp.full((L,), w, jnp.int32) < wid_v, cw, 0)
    valid = lanes < NUM_EXPERTS
    totm = jnp.where(valid, total, 0)
    nt = (totm + (TILE - 1)) // TILE
    tsi = jnp.cumsum(nt)
    tse = tsi - nt
    cursor = tse * TILE + pre
    tt = jnp.sum(nt)

    @pl.when(wid == 0)
    def _():
        for half in range((NUM_TILES + 8) // L):
            tvec = lax.iota(jnp.int32, L) + half * L
            acc = jnp.zeros((L,), jnp.int32)
            for e in range(1, NUM_EXPERTS):
                se = tse.at[jnp.full((L,), e, jnp.int32)].get(
                    mode="promise_in_bounds")
                acc = acc + (tvec >= se).astype(jnp.int32)
            acc = jnp.where(lax.iota(jnp.int32, L) == (NUM_TILES - half * L),
                            jnp.broadcast_to(tt, (L,)), acc)
            m_v[pl.ds(half * L, L)] = acc
        pltpu.sync_copy(m_v, meta)
        pltpu.sync_copy(csums, cs_v)
        tot_s = jnp.zeros((L,), jnp.float32)
        for w in range(NW):
            tot_s = tot_s + cs_v[w]
        d = tot_s * (1.0 / BATCH) - 1.0 / NUM_EXPERTS
        cv = jnp.sum(jnp.where(valid, d * d, 0.0))
        cv_v[...] = jnp.broadcast_to(cv, (L,))
        pltpu.sync_copy(cv_v, cvv)

    pltpu.sync_copy(e12.at[:, pl.ds(t0, TPW)], e12_v)
    for c, pva, pvb in ((0, p0a, p0b), (1, p1a, p1b)):
        for g in range(GRP):
            ev = e12_v[c, pl.ds(g * L, L)]
            base = cursor.at[ev].get(mode="promise_in_bounds")
            rank = jnp.zeros((L,), jnp.int32)
            hist = jnp.zeros((L,), jnp.int32)
            for e in range(NUM_EXPERTS):
                m = ev == jnp.full((L,), e, jnp.int32)
                csum = jnp.cumsum(m.astype(jnp.int32))
                rank = jnp.where(m, csum - 1, rank)
                hist = hist + jnp.where(
                    lanes == jnp.full((L,), e, jnp.int32),
                    jnp.broadcast_to(jnp.sum(m.astype(jnp.int32)), (L,)),
                    jnp.zeros((L,), jnp.int32))
            pv = pva if g < GRP // 2 else pvb
            pv[pl.ds((g % (GRP // 2)) * L, L)] = base + rank
            cursor = cursor + hist
        pltpu.sync_copy(pva, pos.at[c, pl.ds(t0, HALF)])
        pltpu.sync_copy(pvb, pos.at[c, pl.ds(t0 + HALF, HALF)])

    cpx.wait()
    cp0 = pltpu.async_copy(x_v, xs.at[p0a], sem)
    cp1 = pltpu.async_copy(x_v, xs.at[p1a], sem)
    cp0.wait()
    cp1.wait()
    pltpu.sync_copy(xb.at[pl.ds(t0 + HALF, HALF)], x_v)
    cp0 = pltpu.async_copy(x_v, xs.at[p0b], sem)
    cp1 = pltpu.async_copy(x_v, xs.at[p1b], sem)
    cp0.wait()
    cp1.wait()


CHK = 16
NB = TPW // CHK


@functools.partial(
    pl.kernel,
    out_type=jax.ShapeDtypeStruct((BATCH, OUTPUT_DIM), jnp.float32),
    mesh=_SC_MESH,
    compiler_params=pltpu.CompilerParams(needs_layout_passes=False),
    scratch_types=[
        pltpu.VMEM((TOP_K, CHK), jnp.int32),
        pltpu.VMEM((TOP_K, CHK), jnp.int32),
        pltpu.VMEM((TOP_K, TPW), jnp.float32),
        pltpu.VMEM((CHK, OUTPUT_DIM), jnp.float32),
        pltpu.VMEM((CHK, OUTPUT_DIM), jnp.float32),
        pltpu.VMEM((CHK, OUTPUT_DIM), jnp.float32),
        pltpu.VMEM((CHK, OUTPUT_DIM), jnp.float32),
        pltpu.VMEM((CHK, OUTPUT_DIM), jnp.float32),
        pltpu.SemaphoreType.DMA,
        pltpu.SemaphoreType.DMA,
    ],
)
def _combine(ys, pos, wtk, out,
             idx0, idx1, w_v, a0, b0, a1, b1, o_v, sem0, sem1):
    wid = _wid()
    t0 = wid * TPW
    pltpu.sync_copy(wtk.at[:, pl.ds(t0, TPW)], w_v)
    bufs = ((idx0, a0, b0, sem0), (idx1, a1, b1, sem1))

    def fire(q):
        idx, a_v, b_v, sem = bufs[q % 2]
        pltpu.sync_copy(pos.at[0, pl.ds(t0 + q * CHK, CHK)], idx.at[0])
        pltpu.sync_copy(pos.at[1, pl.ds(t0 + q * CHK, CHK)], idx.at[1])
        cpa = pltpu.async_copy(ys.at[idx.at[0]], a_v, sem)
        cpb = pltpu.async_copy(ys.at[idx.at[1]], b_v, sem)
        return cpa, cpb

    pend = fire(0)
    for q in range(NB):
        _, a_v, b_v, _ = bufs[q % 2]
        nxt = None
        pend[0].wait()
        pend[1].wait()
        if q + 1 < NB:
            nxt = fire(q + 1)
        wv0 = w_v[0, pl.ds(q * CHK, CHK)]
        wv1 = w_v[1, pl.ds(q * CHK, CHK)]

        def row_body(r, carry, wv0=wv0, wv1=wv1, a_v=a_v, b_v=b_v):
            rl = jnp.broadcast_to(r, (L,))
            w0s = wv0.at[rl].get(mode="promise_in_bounds")
            w1s = wv1.at[rl].get(mode="promise_in_bounds")
            for k in range(OUTPUT_DIM // L):
                sl = pl.ds(k * L, L)
                o_v[r, sl] = a_v[r, sl] * w0s + b_v[r, sl] * w1s
            return carry

        lax.fori_loop(0, CHK, row_body, 0)
        pltpu.sync_copy(o_v, out.at[pl.ds(t0 + q * CHK, CHK)])
        if nxt is not None:
            pend = nxt


def _ffn_body(meta_ref, xs_ref, w1_ref, w2_ref, b1_ref, b2_ref, ys_ref):
    i = pl.program_id(0)
    nreal = meta_ref[NUM_TILES]

    @pl.when(i < nreal)
    def _():
        h = jnp.dot(xs_ref[...].astype(jnp.bfloat16), w1_ref[0],
                    preferred_element_type=jnp.float32)
        h = jnp.maximum(h + b1_ref[0, 0].astype(jnp.float32), 0.0)
        y = jnp.dot(h.astype(jnp.bfloat16), w2_ref[0],
                    preferred_element_type=jnp.float32)
        ys_ref[...] = y + b2_ref[0, 0].astype(jnp.float32)


def _grouped_ffn(xs, meta, W1b, b1, W2b, b2):
    grid_spec = pltpu.PrefetchScalarGridSpec(
        num_scalar_prefetch=1,
        grid=(NUM_TILES,),
        in_specs=[
            pl.BlockSpec((TILE, INPUT_DIM), lambda i, m: (i, 0)),
            pl.BlockSpec((1, INPUT_DIM, HIDDEN_DIM),
                         lambda i, m: (m[i], 0, 0)),
            pl.BlockSpec((1, HIDDEN_DIM, OUTPUT_DIM),
                         lambda i, m: (m[i], 0, 0)),
            pl.BlockSpec((1, 1, HIDDEN_DIM), lambda i, m: (m[i], 0, 0)),
            pl.BlockSpec((1, 1, OUTPUT_DIM), lambda i, m: (m[i], 0, 0)),
        ],
        out_specs=pl.BlockSpec((TILE, OUTPUT_DIM), lambda i, m: (i, 0)),
    )
    return pl.pallas_call(
        _ffn_body,
        grid_spec=grid_spec,
        out_shape=jax.ShapeDtypeStruct((SLOTS, OUTPUT_DIM), jnp.float32),
        compiler_params=pltpu.CompilerParams(
            dimension_semantics=("arbitrary",)),
    )(meta, xs, W1b, W2b, b1.reshape(NUM_EXPERTS, 1, HIDDEN_DIM),
      b2.reshape(NUM_EXPERTS, 1, OUTPUT_DIM))


def kernel(x, gate_w1, gate_b1, gate_w2, gate_b2, W1, b1, W2, b2):
    gh = jax.nn.relu(x @ gate_w1 + gate_b1)
    logits = gh @ gate_w2 + gate_b2
    gate_weights = jax.nn.softmax(logits, axis=1)

    gwT = gate_weights.T

    e12, wtk, counts, csums = _router_a(gwT)
    pos, meta, cvv, xs = _router_b(e12, counts, csums, x)
    ys = _grouped_ffn(xs, meta, W1.astype(jnp.bfloat16), b1,
                      W2.astype(jnp.bfloat16), b2)
    output = _combine(ys, pos, wtk)
    return (output, gate_weights, cvv[0])


# speedup vs baseline: 1.0647x; 1.0647x over previous
import functools

import jax
import jax.numpy as jnp
from jax import lax
from jax.experimental import pallas as pl
from jax.experimental.pallas import tpu as pltpu
from jax.experimental.pallas import tpu_sc as plsc

INPUT_DIM = 1024
HIDDEN_DIM = 2048
OUTPUT_DIM = 1024
NUM_EXPERTS = 8
TOP_K = 2
BATCH = 4096

TILE = 512
NUM_TILES = (BATCH * TOP_K) // TILE + (NUM_EXPERTS - 1)
NUM_TILES = ((NUM_TILES + 7) // 8) * 8
SLOTS = NUM_TILES * TILE

NC = 2
NS = 16
NW = NC * NS
L = 16
TPW = BATCH // NW
GRP = TPW // L

_SC_MESH = plsc.VectorSubcoreMesh(core_axis_name="c", subcore_axis_name="s")


def _wid():
    return lax.axis_index("s") * NC + lax.axis_index("c")


@functools.partial(
    pl.kernel,
    out_type=(
        jax.ShapeDtypeStruct((TOP_K, BATCH), jnp.int32),
        jax.ShapeDtypeStruct((TOP_K, BATCH), jnp.float32),
        jax.ShapeDtypeStruct((NW, L), jnp.int32),
        jax.ShapeDtypeStruct((NW, L), jnp.float32),
    ),
    mesh=_SC_MESH,
    compiler_params=pltpu.CompilerParams(needs_layout_passes=False),
    scratch_types=[
        pltpu.VMEM((NUM_EXPERTS, TPW), jnp.float32),
        pltpu.VMEM((TPW,), jnp.int32),
        pltpu.VMEM((TPW,), jnp.int32),
        pltpu.VMEM((TPW,), jnp.float32),
        pltpu.VMEM((TPW,), jnp.float32),
        pltpu.VMEM((L,), jnp.int32),
        pltpu.VMEM((L,), jnp.float32),
    ],
)
def _router_a(gwT, e12, wtk, counts, csums,
              gw_v, e1_v, e2_v, w0_v, w1_v, cnt_v, sv_v):
    wid = _wid()
    t0 = wid * TPW
    pltpu.sync_copy(gwT.at[:, pl.ds(t0, TPW)], gw_v)
    lanes = lax.iota(jnp.int32, L)
    cnt = jnp.zeros((L,), jnp.int32)
    sv = jnp.zeros((L,), jnp.float32)
    ones_f = jnp.full((L,), 1.0, jnp.float32)
    neg_f = jnp.full((L,), -1.0, jnp.float32)
    zeros_i = jnp.zeros((L,), jnp.int32)
    zeros_f = jnp.zeros((L,), jnp.float32)
    ev_const = [jnp.full((L,), e, jnp.int32) for e in range(NUM_EXPERTS)]
    for g in range(GRP):
        sl = pl.ds(g * L, L)
        le = [gw_v[e, sl] for e in range(NUM_EXPERTS)]
        m1 = le[0]
        for e in range(1, NUM_EXPERTS):
            m1 = jnp.maximum(m1, le[e])
        e1 = zeros_i
        for e in range(NUM_EXPERTS - 1, -1, -1):
            e1 = jnp.where(le[e] == m1, ev_const[e], e1)
        le2 = [jnp.where(e1 == ev_const[e], neg_f, le[e])
               for e in range(NUM_EXPERTS)]
        m2 = le2[0]
        for e in range(1, NUM_EXPERTS):
            m2 = jnp.maximum(m2, le2[e])
        e2 = zeros_i
        for e in range(NUM_EXPERTS - 1, -1, -1):
            e2 = jnp.where(le2[e] == m2, ev_const[e], e2)
        z = jnp.exp(m2 - m1)
        w0 = ones_f / (ones_f + z)
        e1_v[sl] = e1
        e2_v[sl] = e2
        w0_v[sl] = w0
        w1_v[sl] = z * w0
        for e in range(NUM_EXPERTS):
            c = (jnp.sum((e1 == ev_const[e]).astype(jnp.int32))
                 + jnp.sum((e2 == ev_const[e]).astype(jnp.int32)))
            cnt = cnt + jnp.where(lanes == ev_const[e],
                                  jnp.broadcast_to(c, (L,)), zeros_i)
            sv = sv + jnp.where(lanes == ev_const[e],
                                jnp.broadcast_to(jnp.sum(le[e]), (L,)),
                                zeros_f)
    cnt_v[...] = cnt
    sv_v[...] = sv
    pltpu.sync_copy(e1_v, e12.at[0, pl.ds(t0, TPW)])
    pltpu.sync_copy(e2_v, e12.at[1, pl.ds(t0, TPW)])
    pltpu.sync_copy(w0_v, wtk.at[0, pl.ds(t0, TPW)])
    pltpu.sync_copy(w1_v, wtk.at[1, pl.ds(t0, TPW)])
    pltpu.sync_copy(cnt_v, counts.at[wid])
    pltpu.sync_copy(sv_v, csums.at[wid])


@functools.partial(
    pl.kernel,
    out_type=(
        jax.ShapeDtypeStruct((TOP_K, BATCH), jnp.int32),
        jax.ShapeDtypeStruct((NUM_TILES + 8, ), jnp.int32),
        jax.ShapeDtypeStruct((L,), jnp.float32),
        jax.ShapeDtypeStruct((SLOTS, INPUT_DIM), jnp.float32),
    ),
    mesh=_SC_MESH,
    compiler_params=pltpu.CompilerParams(needs_layout_passes=False),
    scratch_types=[
        pltpu.VMEM((NW, L), jnp.int32),
        pltpu.VMEM((NW, L), jnp.float32),
        pltpu.VMEM((TOP_K, TPW), jnp.int32),
        pltpu.VMEM((TPW // 2,), jnp.int32),
        pltpu.VMEM((TPW // 2,), jnp.int32),
        pltpu.VMEM((TPW // 2,), jnp.int32),
        pltpu.VMEM((TPW // 2,), jnp.int32),
        pltpu.VMEM((TPW // 2, INPUT_DIM), jnp.float32),
        pltpu.VMEM((NUM_TILES + 8,), jnp.int32),
        pltpu.VMEM((L,), jnp.float32),
        pltpu.SemaphoreType.DMA,
    ],
)
def _router_b(e12, counts, csums, xb, pos, meta, cvv, xs,
              cnt_v, cs_v, e12_v, p0a, p0b, p1a, p1b, x_v, m_v, cv_v, sem):
    wid = _wid()
    t0 = wid * TPW
    lanes = lax.iota(jnp.int32, L)
    wid_v = jnp.broadcast_to(wid, (L,))

    HALF = TPW // 2
    cpx = pltpu.async_copy(xb.at[pl.ds(t0, HALF)], x_v, sem)
    pltpu.sync_copy(counts, cnt_v)
    total = jnp.zeros((L,), jnp.int32)
    pre = jnp.zeros((L,), jnp.int32)
    for w in range(NW):
        cw = cnt_v[w]
        total = total + cw
        pre = pre + jnp.where(jnp.full((L,), w, jnp.int32) < wid_v, cw, 0)
    valid = lanes < NUM_EXPERTS
    totm = jnp.where(valid, total, 0)
    nt = (totm + (TILE - 1)) // TILE
    tsi = jnp.cumsum(nt)
    tse = tsi - nt
    cursor = tse * TILE + pre
    tt = jnp.sum(nt)

    @pl.when(wid == 0)
    def _():
        for half in range((NUM_TILES + 8) // L):
            tvec = lax.iota(jnp.int32, L) + half * L
            acc = jnp.zeros((L,), jnp.int32)
            for e in range(1, NUM_EXPERTS):
                se = tse.at[jnp.full((L,), e, jnp.int32)].get(
                    mode="promise_in_bounds")
                acc = acc + (tvec >= se).astype(jnp.int32)
            acc = jnp.where(lax.iota(jnp.int32, L) == (NUM_TILES - half * L),
                            jnp.broadcast_to(tt, (L,)), acc)
            m_v[pl.ds(half * L, L)] = acc
        pltpu.sync_copy(m_v, meta)
        pltpu.sync_copy(csums, cs_v)
        tot_s = jnp.zeros((L,), jnp.float32)
        for w in range(NW):
            tot_s = tot_s + cs_v[w]
        d = tot_s * (1.0 / BATCH) - 1.0 / NUM_EXPERTS
        cv = jnp.sum(jnp.where(valid, d * d, 0.0))
        cv_v[...] = jnp.broadcast_to(cv, (L,))
        pltpu.sync_copy(cv_v, cvv)

    pltpu.sync_copy(e12.at[:, pl.ds(t0, TPW)], e12_v)
    for c, pva, pvb in ((0, p0a, p0b), (1, p1a, p1b)):
        for g in range(GRP):
            ev = e12_v[c, pl.ds(g * L, L)]
            base = cursor.at[ev].get(mode="promise_in_bounds")
            rank = jnp.zeros((L,), jnp.int32)
            hist = jnp.zeros((L,), jnp.int32)
            for e in range(NUM_EXPERTS):
                m = ev == jnp.full((L,), e, jnp.int32)
                csum = jnp.cumsum(m.astype(jnp.int32))
                rank = jnp.where(m, csum - 1, rank)
                hist = hist + jnp.where(
                    lanes == jnp.full((L,), e, jnp.int32),
                    jnp.broadcast_to(jnp.sum(m.astype(jnp.int32)), (L,)),
                    jnp.zeros((L,), jnp.int32))
            pv = pva if g < GRP // 2 else pvb
            pv[pl.ds((g % (GRP // 2)) * L, L)] = base + rank
            cursor = cursor + hist
        pltpu.sync_copy(pva, pos.at[c, pl.ds(t0, HALF)])
        pltpu.sync_copy(pvb, pos.at[c, pl.ds(t0 + HALF, HALF)])

    cpx.wait()
    cp0 = pltpu.async_copy(x_v, xs.at[p0a], sem)
    cp1 = pltpu.async_copy(x_v, xs.at[p1a], sem)
    cp0.wait()
    cp1.wait()
    pltpu.sync_copy(xb.at[pl.ds(t0 + HALF, HALF)], x_v)
    cp0 = pltpu.async_copy(x_v, xs.at[p0b], sem)
    cp1 = pltpu.async_copy(x_v, xs.at[p1b], sem)
    cp0.wait()
    cp1.wait()


CHK = 16
NB = TPW // CHK


@functools.partial(
    pl.kernel,
    out_type=jax.ShapeDtypeStruct((BATCH, OUTPUT_DIM), jnp.float32),
    mesh=_SC_MESH,
    compiler_params=pltpu.CompilerParams(needs_layout_passes=False),
    scratch_types=[
        pltpu.VMEM((TOP_K, TPW), jnp.int32),
        pltpu.VMEM((TOP_K, TPW), jnp.float32),
        pltpu.VMEM((CHK, OUTPUT_DIM), jnp.float32),
        pltpu.VMEM((CHK, OUTPUT_DIM), jnp.float32),
        pltpu.VMEM((CHK, OUTPUT_DIM), jnp.float32),
        pltpu.VMEM((CHK, OUTPUT_DIM), jnp.float32),
        pltpu.VMEM((CHK, OUTPUT_DIM), jnp.float32),
        pltpu.SemaphoreType.DMA,
        pltpu.SemaphoreType.DMA,
    ],
)
def _combine(ys, pos, wtk, out,
             pos_v, w_v, a0, b0, a1, b1, o_v, sem0, sem1):
    wid = _wid()
    t0 = wid * TPW
    pltpu.sync_copy(pos.at[:, pl.ds(t0, TPW)], pos_v)
    pltpu.sync_copy(wtk.at[:, pl.ds(t0, TPW)], w_v)
    bufs = ((a0, b0, sem0), (a1, b1, sem1))

    def fire(q):
        a_v, b_v, sem = bufs[q % 2]
        sl = pl.ds(q * CHK, CHK)
        cpa = pltpu.async_copy(ys.at[pos_v.at[0, sl]], a_v, sem)
        cpb = pltpu.async_copy(ys.at[pos_v.at[1, sl]], b_v, sem)
        return cpa, cpb

    pend = fire(0)
    for q in range(NB):
        a_v, b_v, _ = bufs[q % 2]
        nxt = None
        pend[0].wait()
        pend[1].wait()
        if q + 1 < NB:
            nxt = fire(q + 1)
        wv0 = w_v[0, pl.ds(q * CHK, CHK)]
        wv1 = w_v[1, pl.ds(q * CHK, CHK)]

        def row_body(r, carry, wv0=wv0, wv1=wv1, a_v=a_v, b_v=b_v):
            rl = jnp.broadcast_to(r, (L,))
            w0s = wv0.at[rl].get(mode="promise_in_bounds")
            w1s = wv1.at[rl].get(mode="promise_in_bounds")
            for k in range(OUTPUT_DIM // L):
                sl = pl.ds(k * L, L)
                o_v[r, sl] = a_v[r, sl] * w0s + b_v[r, sl] * w1s
            return carry

        lax.fori_loop(0, CHK, row_body, 0)
        pltpu.sync_copy(o_v, out.at[pl.ds(t0 + q * CHK, CHK)])
        if nxt is not None:
            pend = nxt


def _ffn_body(meta_ref, xs_ref, w1_ref, w2_ref, b1_ref, b2_ref, ys_ref):
    i = pl.program_id(0)
    nreal = meta_ref[NUM_TILES]

    @pl.when(i < nreal)
    def _():
        h = jnp.dot(xs_ref[...].astype(jnp.bfloat16), w1_ref[0],
                    preferred_element_type=jnp.float32)
        h = jnp.maximum(h + b1_ref[0, 0].astype(jnp.float32), 0.0)
        y = jnp.dot(h.astype(jnp.bfloat16), w2_ref[0],
                    preferred_element_type=jnp.float32)
        ys_ref[...] = y + b2_ref[0, 0].astype(jnp.float32)


def _grouped_ffn(xs, meta, W1b, b1, W2b, b2):
    grid_spec = pltpu.PrefetchScalarGridSpec(
        num_scalar_prefetch=1,
        grid=(NUM_TILES,),
        in_specs=[
            pl.BlockSpec((TILE, INPUT_DIM), lambda i, m: (i, 0)),
            pl.BlockSpec((1, INPUT_DIM, HIDDEN_DIM),
                         lambda i, m: (m[i], 0, 0)),
            pl.BlockSpec((1, HIDDEN_DIM, OUTPUT_DIM),
                         lambda i, m: (m[i], 0, 0)),
            pl.BlockSpec((1, 1, HIDDEN_DIM), lambda i, m: (m[i], 0, 0)),
            pl.BlockSpec((1, 1, OUTPUT_DIM), lambda i, m: (m[i], 0, 0)),
        ],
        out_specs=pl.BlockSpec((TILE, OUTPUT_DIM), lambda i, m: (i, 0)),
    )
    return pl.pallas_call(
        _ffn_body,
        grid_spec=grid_spec,
        out_shape=jax.ShapeDtypeStruct((SLOTS, OUTPUT_DIM), jnp.float32),
        compiler_params=pltpu.CompilerParams(
            dimension_semantics=("arbitrary",)),
    )(meta, xs, W1b, W2b, b1.reshape(NUM_EXPERTS, 1, HIDDEN_DIM),
      b2.reshape(NUM_EXPERTS, 1, OUTPUT_DIM))


def kernel(x, gate_w1, gate_b1, gate_w2, gate_b2, W1, b1, W2, b2):
    gh = jax.nn.relu(x @ gate_w1 + gate_b1)
    logits = gh @ gate_w2 + gate_b2
    gate_weights = jax.nn.softmax(logits, axis=1)

    gwT = gate_weights.T

    e12, wtk, counts, csums = _router_a(gwT)
    pos, meta, cvv, xs = _router_b(e12, counts, csums, x)
    ys = _grouped_ffn(xs, meta, W1.astype(jnp.bfloat16), b1,
                      W2.astype(jnp.bfloat16), b2)
    output = _combine(ys, pos, wtk)
    return (output, gate_weights, cvv[0])
